# baseline (speedup 1.0000x reference)
import functools

import jax
import jax.numpy as jnp
from jax import lax
from jax.experimental import pallas as pl
from jax.experimental.pallas import tpu as pltpu
from jax.experimental.pallas import tpu_sc as plsc

B, S, D = 64, 2048, 512
TOPK_N = 8
SCALE = 1.0 / (S * D)

B_TC = 48
NSC = B - B_TC
NW = 32
RPT = S // NW
CH_ROWS = 8
NCH = RPT // CH_ROWS
T_CH = NSC * NCH

NSPLIT = 8
ROWS = S // NSPLIT
SPB = 2


def _mse_body(*refs):
    o_refs, l_refs, out_ref = refs[:NSPLIT], refs[NSPLIT:-1], refs[-1]
    i = pl.program_id(0)
    acc = jnp.zeros((SPB, 8, 128), jnp.float32)
    for o_ref, l_ref in zip(o_refs, l_refs):
        d = (o_ref[...] - l_ref[...]).reshape(SPB, -1, 8, 128)
        acc = acc + jnp.sum(d * d, axis=1)
    for s in range(SPB):
        out_ref[i * SPB + s] = jnp.sum(acc[s]) * SCALE


def _tc_losses(output, label):
    in_specs = [
        pl.BlockSpec((SPB, ROWS, D), lambda i, j=j: (i, j, 0))
        for j in range(NSPLIT)
    ]
    out_spec = pl.BlockSpec(memory_space=pltpu.SMEM)
    return pl.pallas_call(
        _mse_body,
        grid=(B_TC // SPB,),
        in_specs=in_specs + in_specs,
        out_specs=out_spec,
        out_shape=jax.ShapeDtypeStruct((B_TC,), jnp.float32),
    )(*([output] * NSPLIT), *([label] * NSPLIT))


def _vsort(x):
    k, _ = plsc.sort_key_val(x, x)
    return k


def _merge_top(a, b):
    return _vsort(jnp.maximum(a, jnp.flip(b, 0)))


@functools.cache
def _make_sc_partials():
    @functools.partial(
        pl.kernel,
        out_type=jax.ShapeDtypeStruct((NW, 16), jnp.float32),
        mesh=plsc.VectorSubcoreMesh(core_axis_name="c", subcore_axis_name="s"),
        compiler_params=pltpu.CompilerParams(needs_layout_passes=False),
        scratch_types=[
            pltpu.VMEM((2, CH_ROWS, D), jnp.float32),
            pltpu.VMEM((2, CH_ROWS, D), jnp.float32),
            pltpu.VMEM((16,), jnp.float32),
            pltpu.SemaphoreType.DMA,
            pltpu.SemaphoreType.DMA,
            pltpu.SemaphoreType.DMA,
            pltpu.SemaphoreType.DMA,
        ],
    )
    def _sc_partials(of_hbm, lf_hbm, out_hbm, o_buf, l_buf, part_v,
                     so0, so1, sl0, sl1):
        so = (so0, so1)
        sl = (sl0, sl1)
        wid = lax.axis_index("s") * 2 + lax.axis_index("c")
        base_row = wid * RPT

        def _row_of(t):
            b = t // NCH
            c = t % NCH
            return (B_TC + b) * S + base_row + c * CH_ROWS

        def _start(t, slot):
            row = _row_of(t)
            pltpu.async_copy(
                of_hbm.at[pl.ds(row, CH_ROWS)], o_buf.at[slot], so[slot])
            pltpu.async_copy(
                lf_hbm.at[pl.ds(row, CH_ROWS)], l_buf.at[slot], sl[slot])

        def _wait(slot):
            pltpu.make_async_copy(
                of_hbm.at[pl.ds(0, CH_ROWS)], o_buf.at[slot], so[slot]).wait()
            pltpu.make_async_copy(
                lf_hbm.at[pl.ds(0, CH_ROWS)], l_buf.at[slot], sl[slot]).wait()

        _start(0, 0)
        _start(1, 1)

        def _chunk_acc(slot, acc):
            def _r_body(r, acc):
                for j in range(D // 16):
                    o = o_buf[slot, r, pl.ds(16 * j, 16)]
                    l = l_buf[slot, r, pl.ds(16 * j, 16)]
                    d = o - l
                    acc = acc + d * d
                return acc

            return lax.fori_loop(0, CH_ROWS, _r_body, acc)

        lane = lax.broadcasted_iota(jnp.int32, (16,), 0)

        def _b_body(b, total_vec):
            acc = jnp.zeros((16,), jnp.float32)
            for c in range(NCH):
                slot = c % 2
                t = b * NCH + c
                _wait(slot)
                acc = _chunk_acc(slot, acc)

                @pl.when(t + 2 < T_CH)
                def _():
                    _start(t + 2, slot)

            return total_vec + jnp.where(
                lane == b, jnp.sum(acc) * SCALE, jnp.float32(0.0))

        total_vec = lax.fori_loop(
            0, NSC, _b_body, jnp.zeros((16,), jnp.float32))
        part_v[...] = total_vec
        pltpu.sync_copy(part_v, out_hbm.at[wid])

    return _sc_partials


@functools.cache
def _make_sc_merge_topk():
    @functools.partial(
        pl.kernel,
        out_type=jax.ShapeDtypeStruct((16,), jnp.float32),
        mesh=plsc.VectorSubcoreMesh(core_axis_name="c", subcore_axis_name="s"),
        compiler_params=pltpu.CompilerParams(needs_layout_passes=False),
        scratch_types=[
            pltpu.VMEM((B_TC,), jnp.float32),
            pltpu.VMEM((NW, 16), jnp.float32),
            pltpu.VMEM((16,), jnp.float32),
        ],
    )
    def _sc_merge(tcl_hbm, parts_hbm, out_hbm, tcl_v, parts_v, out_v):
        cid = lax.axis_index("c")
        sid = lax.axis_index("s")

        @pl.when((cid == 0) & (sid == 0))
        def _():
            pltpu.sync_copy(tcl_hbm, tcl_v)
            pltpu.sync_copy(parts_hbm, parts_v)
            ssc = parts_v[0, pl.ds(0, 16)]
            for w in range(1, NW):
                ssc = ssc + parts_v[w, pl.ds(0, 16)]
            s0 = _vsort(tcl_v[pl.ds(0, 16)])
            s1 = _vsort(tcl_v[pl.ds(16, 16)])
            s2 = _vsort(tcl_v[pl.ds(32, 16)])
            s3 = _vsort(ssc)
            top = _merge_top(_merge_top(s0, s1), _merge_top(s2, s3))
            out_v[...] = jnp.flip(top, 0)
            pltpu.sync_copy(out_v, out_hbm)

    return _sc_merge


def kernel(output, label):
    of = output.reshape(B * S, D)
    lf = label.reshape(B * S, D)
    losses_tc = _tc_losses(output, label)
    partials = _make_sc_partials()(of, lf)
    top16_desc = _make_sc_merge_topk()(losses_tc, partials)
    return top16_desc[:TOPK_N]


# speedup vs baseline: 1.0061x; 1.0061x over previous
import functools

import jax
import jax.numpy as jnp
from jax import lax
from jax.experimental import pallas as pl
from jax.experimental.pallas import tpu as pltpu
from jax.experimental.pallas import tpu_sc as plsc

B, S, D = 64, 2048, 512
TOPK_N = 8
SCALE = 1.0 / (S * D)

B_TC = 48
NSC = B - B_TC
NW = 32
RPT = S // NW
CH_ROWS = 16
NCH = RPT // CH_ROWS
T_CH = NSC * NCH

NSPLIT = 8
ROWS = S // NSPLIT
SPB = 2


def _mse_body(*refs):
    o_refs, l_refs, out_ref = refs[:NSPLIT], refs[NSPLIT:-1], refs[-1]
    i = pl.program_id(0)
    acc = jnp.zeros((SPB, 8, 128), jnp.float32)
    for o_ref, l_ref in zip(o_refs, l_refs):
        d = (o_ref[...] - l_ref[...]).reshape(SPB, -1, 8, 128)
        acc = acc + jnp.sum(d * d, axis=1)
    for s in range(SPB):
        out_ref[i * SPB + s] = jnp.sum(acc[s]) * SCALE


def _tc_losses(output, label):
    in_specs = [
        pl.BlockSpec((SPB, ROWS, D), lambda i, j=j: (i, j, 0))
        for j in range(NSPLIT)
    ]
    out_spec = pl.BlockSpec(memory_space=pltpu.SMEM)
    return pl.pallas_call(
        _mse_body,
        grid=(B_TC // SPB,),
        in_specs=in_specs + in_specs,
        out_specs=out_spec,
        out_shape=jax.ShapeDtypeStruct((B_TC,), jnp.float32),
    )(*([output] * NSPLIT), *([label] * NSPLIT))


def _vsort(x):
    k, _ = plsc.sort_key_val(x, x)
    return k


def _merge_top(a, b):
    return _vsort(jnp.maximum(a, jnp.flip(b, 0)))


@functools.cache
def _make_sc_partials():
    @functools.partial(
        pl.kernel,
        out_type=jax.ShapeDtypeStruct((NW, 16), jnp.float32),
        mesh=plsc.VectorSubcoreMesh(core_axis_name="c", subcore_axis_name="s"),
        compiler_params=pltpu.CompilerParams(needs_layout_passes=False),
        scratch_types=[
            pltpu.VMEM((2, CH_ROWS, D), jnp.float32),
            pltpu.VMEM((2, CH_ROWS, D), jnp.float32),
            pltpu.VMEM((16,), jnp.float32),
            pltpu.SemaphoreType.DMA,
            pltpu.SemaphoreType.DMA,
            pltpu.SemaphoreType.DMA,
            pltpu.SemaphoreType.DMA,
        ],
    )
    def _sc_partials(of_hbm, lf_hbm, out_hbm, o_buf, l_buf, part_v,
                     so0, so1, sl0, sl1):
        so = (so0, so1)
        sl = (sl0, sl1)
        wid = lax.axis_index("s") * 2 + lax.axis_index("c")
        base_row = wid * RPT

        def _row_of(t):
            b = t // NCH
            c = t % NCH
            return (B_TC + b) * S + base_row + c * CH_ROWS

        def _start(t, slot):
            row = _row_of(t)
            pltpu.async_copy(
                of_hbm.at[pl.ds(row, CH_ROWS)], o_buf.at[slot], so[slot])
            pltpu.async_copy(
                lf_hbm.at[pl.ds(row, CH_ROWS)], l_buf.at[slot], sl[slot])

        def _wait(slot):
            pltpu.make_async_copy(
                of_hbm.at[pl.ds(0, CH_ROWS)], o_buf.at[slot], so[slot]).wait()
            pltpu.make_async_copy(
                lf_hbm.at[pl.ds(0, CH_ROWS)], l_buf.at[slot], sl[slot]).wait()

        _start(0, 0)
        _start(1, 1)

        def _chunk_acc(slot, acc):
            def _r_body(r, acc):
                for j in range(D // 16):
                    o = o_buf[slot, r, pl.ds(16 * j, 16)]
                    l = l_buf[slot, r, pl.ds(16 * j, 16)]
                    d = o - l
                    acc = acc + d * d
                return acc

            return lax.fori_loop(0, CH_ROWS, _r_body, acc)

        lane = lax.broadcasted_iota(jnp.int32, (16,), 0)

        def _b_body(b, total_vec):
            acc = jnp.zeros((16,), jnp.float32)
            for c in range(NCH):
                slot = c % 2
                t = b * NCH + c
                _wait(slot)
                acc = _chunk_acc(slot, acc)

                @pl.when(t + 2 < T_CH)
                def _():
                    _start(t + 2, slot)

            return total_vec + jnp.where(
                lane == b, jnp.sum(acc) * SCALE, jnp.float32(0.0))

        total_vec = lax.fori_loop(
            0, NSC, _b_body, jnp.zeros((16,), jnp.float32))
        part_v[...] = total_vec
        pltpu.sync_copy(part_v, out_hbm.at[wid])

    return _sc_partials


@functools.cache
def _make_sc_merge_topk():
    @functools.partial(
        pl.kernel,
        out_type=jax.ShapeDtypeStruct((16,), jnp.float32),
        mesh=plsc.VectorSubcoreMesh(core_axis_name="c", subcore_axis_name="s"),
        compiler_params=pltpu.CompilerParams(needs_layout_passes=False),
        scratch_types=[
            pltpu.VMEM((B_TC,), jnp.float32),
            pltpu.VMEM((NW, 16), jnp.float32),
            pltpu.VMEM((16,), jnp.float32),
        ],
    )
    def _sc_merge(tcl_hbm, parts_hbm, out_hbm, tcl_v, parts_v, out_v):
        cid = lax.axis_index("c")
        sid = lax.axis_index("s")

        @pl.when((cid == 0) & (sid == 0))
        def _():
            pltpu.sync_copy(tcl_hbm, tcl_v)
            pltpu.sync_copy(parts_hbm, parts_v)
            ssc = parts_v[0, pl.ds(0, 16)]
            for w in range(1, NW):
                ssc = ssc + parts_v[w, pl.ds(0, 16)]
            s0 = _vsort(tcl_v[pl.ds(0, 16)])
            s1 = _vsort(tcl_v[pl.ds(16, 16)])
            s2 = _vsort(tcl_v[pl.ds(32, 16)])
            s3 = _vsort(ssc)
            top = _merge_top(_merge_top(s0, s1), _merge_top(s2, s3))
            out_v[...] = jnp.flip(top, 0)
            pltpu.sync_copy(out_v, out_hbm)

    return _sc_merge


def kernel(output, label):
    of = output.reshape(B * S, D)
    lf = label.reshape(B * S, D)
    losses_tc = _tc_losses(output, label)
    partials = _make_sc_partials()(of, lf)
    top16_desc = _make_sc_merge_topk()(losses_tc, partials)
    return top16_desc[:TOPK_N]


# speedup vs baseline: 1.0311x; 1.0248x over previous
import functools

import jax
import jax.numpy as jnp
from jax import lax
from jax.experimental import pallas as pl
from jax.experimental.pallas import tpu as pltpu
from jax.experimental.pallas import tpu_sc as plsc

B, S, D = 64, 2048, 512
TOPK_N = 8
SCALE = 1.0 / (S * D)

NSPLIT = 8
ROWS = S // NSPLIT
SPB = 2


def _mse_body(*refs):
    o_refs, l_refs, out_ref = refs[:NSPLIT], refs[NSPLIT:-1], refs[-1]
    i = pl.program_id(0)
    acc = jnp.zeros((SPB, 8, 128), jnp.float32)
    for o_ref, l_ref in zip(o_refs, l_refs):
        d = (o_ref[...] - l_ref[...]).reshape(SPB, -1, 8, 128)
        acc = acc + jnp.sum(d * d, axis=1)
    for s in range(SPB):
        out_ref[i * SPB + s] = jnp.sum(acc[s]) * SCALE


def _per_sample_mse(output, label):
    in_specs = [
        pl.BlockSpec((SPB, ROWS, D), lambda i, j=j: (i, j, 0))
        for j in range(NSPLIT)
    ]
    out_spec = pl.BlockSpec(memory_space=pltpu.SMEM)
    return pl.pallas_call(
        _mse_body,
        grid=(B // SPB,),
        in_specs=in_specs + in_specs,
        out_specs=out_spec,
        out_shape=jax.ShapeDtypeStruct((B,), jnp.float32),
    )(*([output] * NSPLIT), *([label] * NSPLIT))


def _vsort(x):
    k, _ = plsc.sort_key_val(x, x)
    return k


def _merge_top(a, b):
    return _vsort(jnp.maximum(a, jnp.flip(b, 0)))


@functools.cache
def _make_sc_top16():
    @functools.partial(
        pl.kernel,
        out_type=jax.ShapeDtypeStruct((16,), jnp.float32),
        mesh=plsc.VectorSubcoreMesh(core_axis_name="c", subcore_axis_name="s"),
        compiler_params=pltpu.CompilerParams(needs_layout_passes=False),
        scratch_types=[
            pltpu.VMEM((B,), jnp.float32),
            pltpu.VMEM((16,), jnp.float32),
        ],
    )
    def _sc_top16(losses_hbm, out_hbm, vals_v, out_v):
        cid = lax.axis_index("c")
        sid = lax.axis_index("s")

        @pl.when((cid == 0) & (sid == 0))
        def _():
            pltpu.sync_copy(losses_hbm, vals_v)
            s0 = _vsort(vals_v[pl.ds(0, 16)])
            s1 = _vsort(vals_v[pl.ds(16, 16)])
            s2 = _vsort(vals_v[pl.ds(32, 16)])
            s3 = _vsort(vals_v[pl.ds(48, 16)])
            top = _merge_top(_merge_top(s0, s1), _merge_top(s2, s3))
            out_v[...] = jnp.flip(top, 0)
            pltpu.sync_copy(out_v, out_hbm)

    return _sc_top16


def kernel(output, label):
    losses = _per_sample_mse(output, label)
    top16_desc = _make_sc_top16()(losses)
    return top16_desc[:TOPK_N]


# speedup vs baseline: 1.0362x; 1.0050x over previous
import functools

import jax
import jax.numpy as jnp
from jax import lax
from jax.experimental import pallas as pl
from jax.experimental.pallas import tpu as pltpu
from jax.experimental.pallas import tpu_sc as plsc

B, S, D = 64, 2048, 512
TOPK_N = 8
SCALE = 1.0 / (S * D)

NSPLIT = 8
ROWS = S // NSPLIT
SPB = 2


def _mse_body(*refs):
    o_refs, l_refs, out_ref = refs[:NSPLIT], refs[NSPLIT:-1], refs[-1]
    i = pl.program_id(0)
    acc = jnp.zeros((SPB, 8, 128), jnp.float32)
    for o_ref, l_ref in zip(o_refs, l_refs):
        d = (o_ref[...] - l_ref[...]).reshape(SPB, -1, 8, 128)
        acc = acc + jnp.sum(d * d, axis=1)
    for s in range(SPB):
        out_ref[i * SPB + s] = jnp.sum(acc[s]) * SCALE


def _per_sample_mse(output, label):
    in_specs = [
        pl.BlockSpec((SPB, ROWS, D), lambda i, j=j: (i, j, 0))
        for j in range(NSPLIT)
    ]
    out_spec = pl.BlockSpec(memory_space=pltpu.SMEM)
    return pl.pallas_call(
        _mse_body,
        grid=(B // SPB,),
        in_specs=in_specs + in_specs,
        out_specs=out_spec,
        out_shape=jax.ShapeDtypeStruct((B,), jnp.float32),
    )(*([output] * NSPLIT), *([label] * NSPLIT))


def _vsort(x):
    k, _ = plsc.sort_key_val(x, x)
    return k


def _merge_top(a, b):
    return _vsort(jnp.maximum(a, jnp.flip(b, 0)))


@functools.cache
def _make_sc_top16():
    @functools.partial(
        pl.kernel,
        out_type=jax.ShapeDtypeStruct((16,), jnp.float32),
        mesh=plsc.VectorSubcoreMesh(core_axis_name="c", subcore_axis_name="s"),
        compiler_params=pltpu.CompilerParams(
            needs_layout_passes=False, skip_device_barrier=True),
        scratch_types=[
            pltpu.VMEM((B,), jnp.float32),
            pltpu.VMEM((16,), jnp.float32),
        ],
    )
    def _sc_top16(losses_hbm, out_hbm, vals_v, out_v):
        cid = lax.axis_index("c")
        sid = lax.axis_index("s")

        @pl.when((cid == 0) & (sid == 0))
        def _():
            pltpu.sync_copy(losses_hbm, vals_v)
            s0 = _vsort(vals_v[pl.ds(0, 16)])
            s1 = _vsort(vals_v[pl.ds(16, 16)])
            s2 = _vsort(vals_v[pl.ds(32, 16)])
            s3 = _vsort(vals_v[pl.ds(48, 16)])
            top = _merge_top(_merge_top(s0, s1), _merge_top(s2, s3))
            out_v[...] = jnp.flip(top, 0)
            pltpu.sync_copy(out_v, out_hbm)

    return _sc_top16


def kernel(output, label):
    losses = _per_sample_mse(output, label)
    top16_desc = _make_sc_top16()(losses)
    return top16_desc[:TOPK_N]


# speedup vs baseline: 1.0437x; 1.0072x over previous
import functools

import jax
import jax.numpy as jnp
from jax import lax
from jax.experimental import pallas as pl
from jax.experimental.pallas import tpu as pltpu
from jax.experimental.pallas import tpu_sc as plsc

B, S, D = 64, 2048, 512
TOPK_N = 8
SCALE = 1.0 / (S * D)

NSPLIT = 8
ROWS = S // NSPLIT
SPB = 2


def _mse_body(*refs):
    o_refs, l_refs, out_ref = refs[:NSPLIT], refs[NSPLIT:-1], refs[-1]
    i = pl.program_id(0)
    acc = jnp.zeros((SPB, 8, 128), jnp.float32)
    for o_ref, l_ref in zip(o_refs, l_refs):
        d = (o_ref[...] - l_ref[...]).reshape(SPB, -1, 8, 128)
        acc = acc + jnp.sum(d * d, axis=1)
    for s in range(SPB):
        out_ref[i * SPB + s] = jnp.sum(acc[s]) * SCALE


def _per_sample_mse(output, label):
    in_specs = [
        pl.BlockSpec((SPB, ROWS, D), lambda i, j=j: (i, j, 0))
        for j in range(NSPLIT)
    ]
    out_spec = pl.BlockSpec(memory_space=pltpu.SMEM)
    return pl.pallas_call(
        _mse_body,
        grid=(B // SPB,),
        in_specs=in_specs + in_specs,
        out_specs=out_spec,
        out_shape=jax.ShapeDtypeStruct((B,), jnp.float32),
    )(*([output] * NSPLIT), *([label] * NSPLIT))


def _vsort(x):
    k, _ = plsc.sort_key_val(x, x)
    return k


def _merge_top(a, b):
    return _vsort(jnp.maximum(a, jnp.flip(b, 0)))


@functools.cache
def _make_sc_top16():
    @functools.partial(
        pl.kernel,
        out_type=jax.ShapeDtypeStruct((16,), jnp.float32),
        mesh=plsc.VectorSubcoreMesh(
            core_axis_name="c", subcore_axis_name="s", num_cores=1),
        compiler_params=pltpu.CompilerParams(
            needs_layout_passes=False, skip_device_barrier=True),
        scratch_types=[
            pltpu.VMEM((B,), jnp.float32),
            pltpu.VMEM((16,), jnp.float32),
        ],
    )
    def _sc_top16(losses_hbm, out_hbm, vals_v, out_v):
        cid = lax.axis_index("c")
        sid = lax.axis_index("s")

        @pl.when((cid == 0) & (sid == 0))
        def _():
            pltpu.sync_copy(losses_hbm, vals_v)
            s0 = _vsort(vals_v[pl.ds(0, 16)])
            s1 = _vsort(vals_v[pl.ds(16, 16)])
            s2 = _vsort(vals_v[pl.ds(32, 16)])
            s3 = _vsort(vals_v[pl.ds(48, 16)])
            top = _merge_top(_merge_top(s0, s1), _merge_top(s2, s3))
            out_v[...] = jnp.flip(top, 0)
            pltpu.sync_copy(out_v, out_hbm)

    return _sc_top16


def kernel(output, label):
    losses = _per_sample_mse(output, label)
    top16_desc = _make_sc_top16()(losses)
    return top16_desc[:TOPK_N]
